# baseline (speedup 1.0000x reference)
import jax
import jax.numpy as jnp
from jax import lax
from jax.experimental import pallas as pl
from jax.experimental.pallas import tpu as pltpu
from jax.experimental.pallas import tpu_sc as plsc

B = 16384
D = 64
L = 16
NC = 2
NS = 16
NW = NC * NS
BPW = B // NW
C = 128
NCHUNK = BPW // C
NGRP = D // L


def _sc_body(h2_hbm, t2_hbm, r_hbm, hp_hbm, tp_hbm,
             ent1_hbm, ent2_hbm, rel_hbm,
             out_hbm,
             hv, tv, rv, hpv, tpv, e1h, e2h, e1t, e2t, rlc, resc, sem):
    wid = lax.axis_index("s") * NC + lax.axis_index("c")

    for chunk in range(NCHUNK):
        base = wid * BPW + chunk * C
        pltpu.sync_copy(h2_hbm.at[pl.ds(base, C)], hv)
        pltpu.sync_copy(t2_hbm.at[pl.ds(base, C)], tv)
        pltpu.sync_copy(r_hbm.at[pl.ds(base, C)], rv)
        pltpu.sync_copy(hp_hbm.at[pl.ds(base, C)], hpv.at[pl.ds(0, C)])
        pltpu.sync_copy(tp_hbm.at[pl.ds(base, C)], tpv.at[pl.ds(0, C)])

        cps = (
            pltpu.make_async_copy(ent1_hbm.at[hv], e1h, sem),
            pltpu.make_async_copy(ent2_hbm.at[hv], e2h, sem),
            pltpu.make_async_copy(ent1_hbm.at[tv], e1t, sem),
            pltpu.make_async_copy(ent2_hbm.at[tv], e2t, sem),
            pltpu.make_async_copy(rel_hbm.at[rv], rlc, sem),
        )
        for cp in cps:
            cp.start()
        for cp in cps:
            cp.wait()

        def e_body(e, carry):
            ph = hpv[pl.ds(e, L)][0] * D
            pt = tpv[pl.ds(e, L)][0] * D
            acc = jnp.zeros((L,), jnp.float32)
            for g in range(NGRP):
                a1 = e1h[e, pl.ds(ph + g * L, L)]
                a2 = e2h[e, pl.ds(ph + g * L, L)]
                b1 = e1t[e, pl.ds(pt + g * L, L)]
                b2 = e2t[e, pl.ds(pt + g * L, L)]
                q1 = rlc[e, pl.ds(g * L, L)]
                q2 = rlc[e, pl.ds(D + g * L, L)]
                acc = acc + q1 * (a1 * b1 + a2 * b2) + q2 * (a1 * b2 - a2 * b1)
            resc[e, :] = acc
            return carry

        lax.fori_loop(0, C, e_body, 0)
        pltpu.sync_copy(resc, out_hbm.at[pl.ds(base, C)])


def _make_sc_kernel():
    mesh = plsc.VectorSubcoreMesh(core_axis_name="c", subcore_axis_name="s")
    return pl.kernel(
        _sc_body,
        out_type=jax.ShapeDtypeStruct((B, L), jnp.float32),
        mesh=mesh,
        compiler_params=pltpu.CompilerParams(use_tc_tiling_on_sc=True),
        scratch_types=[
            pltpu.VMEM((C,), jnp.int32),
            pltpu.VMEM((C,), jnp.int32),
            pltpu.VMEM((C,), jnp.int32),
            pltpu.VMEM((C + L,), jnp.int32),
            pltpu.VMEM((C + L,), jnp.int32),
            pltpu.VMEM((C, 2 * D), jnp.float32),
            pltpu.VMEM((C, 2 * D), jnp.float32),
            pltpu.VMEM((C, 2 * D), jnp.float32),
            pltpu.VMEM((C, 2 * D), jnp.float32),
            pltpu.VMEM((C, 2 * D), jnp.float32),
            pltpu.VMEM((C, L), jnp.float32),
            pltpu.SemaphoreType.DMA,
        ],
    )


def _loss_body(res_ref, y_ref, out_ref):
    s = jnp.sum(res_ref[...], axis=2)
    out_ref[0, 0] = jnp.mean(jax.nn.softplus(-y_ref[...] * s))


@jax.jit
def kernel(h, t, r, y, ent1, ent2, rel1, rel2):
    h = h.astype(jnp.int32)
    t = t.astype(jnp.int32)
    r = r.astype(jnp.int32)
    ent1v = ent1.reshape(ent1.shape[0] // 2, 2 * D)
    ent2v = ent2.reshape(ent2.shape[0] // 2, 2 * D)
    relv = jnp.concatenate([rel1, rel2], axis=1)
    partial = _make_sc_kernel()(
        h >> 1, t >> 1, r, h & 1, t & 1, ent1v, ent2v, relv)
    loss = pl.pallas_call(
        _loss_body,
        out_shape=jax.ShapeDtypeStruct((1, 1), jnp.float32),
        out_specs=pl.BlockSpec(memory_space=pltpu.SMEM),
    )(partial.reshape(128, 128, L), y.reshape(128, 128))
    return loss[0, 0]


# speedup vs baseline: 1.2253x; 1.2253x over previous
import jax
import jax.numpy as jnp
from jax import lax
from jax.experimental import pallas as pl
from jax.experimental.pallas import tpu as pltpu
from jax.experimental.pallas import tpu_sc as plsc

B = 16384
D = 64
L = 16
NC = 2
NS = 16
NW = NC * NS
BPW = B // NW
C = 128
NCHUNK = BPW // C
NGRP = D // L


def _sc_body(h_hbm, t_hbm, r_hbm,
             ent_hbm, rel_hbm,
             out_hbm,
             hv, tv, rv, eh, et, rlc, resc, sem):
    wid = lax.axis_index("s") * NC + lax.axis_index("c")

    for chunk in range(NCHUNK):
        base = wid * BPW + chunk * C
        pltpu.sync_copy(h_hbm.at[pl.ds(base, C)], hv)
        pltpu.sync_copy(t_hbm.at[pl.ds(base, C)], tv)
        pltpu.sync_copy(r_hbm.at[pl.ds(base, C)], rv)

        cps = (
            pltpu.make_async_copy(ent_hbm.at[hv], eh, sem),
            pltpu.make_async_copy(ent_hbm.at[tv], et, sem),
            pltpu.make_async_copy(rel_hbm.at[rv], rlc, sem),
        )
        for cp in cps:
            cp.start()
        for cp in cps:
            cp.wait()

        def e_body(e, carry):
            acc = jnp.zeros((L,), jnp.float32)
            for g in range(NGRP):
                a1 = eh[e, pl.ds(g * L, L)]
                a2 = eh[e, pl.ds(D + g * L, L)]
                b1 = et[e, pl.ds(g * L, L)]
                b2 = et[e, pl.ds(D + g * L, L)]
                q1 = rlc[e, pl.ds(g * L, L)]
                q2 = rlc[e, pl.ds(D + g * L, L)]
                acc = acc + q1 * (a1 * b1 + a2 * b2) + q2 * (a1 * b2 - a2 * b1)
            resc[e, :] = acc
            return carry

        lax.fori_loop(0, C, e_body, 0)
        pltpu.sync_copy(resc, out_hbm.at[pl.ds(base, C)])


def _make_sc_kernel():
    mesh = plsc.VectorSubcoreMesh(core_axis_name="c", subcore_axis_name="s")
    return pl.kernel(
        _sc_body,
        out_type=jax.ShapeDtypeStruct((B, L), jnp.float32),
        mesh=mesh,
        compiler_params=pltpu.CompilerParams(use_tc_tiling_on_sc=True),
        scratch_types=[
            pltpu.VMEM((C,), jnp.int32),
            pltpu.VMEM((C,), jnp.int32),
            pltpu.VMEM((C,), jnp.int32),
            pltpu.VMEM((C, 2 * D), jnp.float32),
            pltpu.VMEM((C, 2 * D), jnp.float32),
            pltpu.VMEM((C, 2 * D), jnp.float32),
            pltpu.VMEM((C, L), jnp.float32),
            pltpu.SemaphoreType.DMA,
        ],
    )


def _loss_body(res_ref, y_ref, out_ref):
    s = jnp.sum(res_ref[...], axis=2)
    out_ref[0, 0] = jnp.mean(jax.nn.softplus(-y_ref[...] * s))


@jax.jit
def kernel(h, t, r, y, ent1, ent2, rel1, rel2):
    h = h.astype(jnp.int32)
    t = t.astype(jnp.int32)
    r = r.astype(jnp.int32)
    entv = jnp.concatenate([ent1, ent2], axis=1)
    relv = jnp.concatenate([rel1, rel2], axis=1)
    partial = _make_sc_kernel()(h, t, r, entv, relv)
    loss = pl.pallas_call(
        _loss_body,
        out_shape=jax.ShapeDtypeStruct((1, 1), jnp.float32),
        out_specs=pl.BlockSpec(memory_space=pltpu.SMEM),
    )(partial.reshape(128, 128, L), y.reshape(128, 128))
    return loss[0, 0]
